# Initial kernel scaffold; baseline (speedup 1.0000x reference)
import jax
import jax.numpy as jnp
from jax.experimental import pallas as pl


def kernel(F_, X, edge_index, Wf1, bf1, Wh1, bh1, Wf2, bf2, Wh2, bh2, Wout, bout):
    raise NotImplementedError("write your pallas kernel here")


# speedup vs baseline: 5.0231x; 5.0231x over previous
import functools

import jax
import jax.numpy as jnp
from jax import lax
from jax.experimental import pallas as pl
from jax.experimental.pallas import tpu as pltpu
from jax.experimental.pallas import tpu_sc as plsc

N = 10000
D = 128
C = 64
E = 320000

_NC = 2
_NS = 16
_NW = _NC * _NS
_EPW = E // _NW
_CHUNK = 80
_NCHUNK = _EPW // _CHUNK
_RPT = 624
_ZR = 16


def _sc_scatter_body(xw_hbm, src_hbm, dst_hbm, out_hbm,
                     src_v, dst_v, rows_v, zbuf, acc_sh, sem):
    c = lax.axis_index("c")
    s = lax.axis_index("s")
    wid = c * _NS + s

    zeros16 = jnp.zeros((16,), jnp.float32)

    def _zb(i, carry):
        zbuf[i // 8, pl.ds((i % 8) * 16, 16)] = zeros16
        return carry

    lax.fori_loop(0, _ZR * 8, _zb, 0)

    def _zc(k, carry):
        pltpu.sync_copy(zbuf, acc_sh.at[pl.ds(s * _RPT + k * _ZR, _ZR)])
        return carry

    lax.fori_loop(0, _RPT // _ZR, _zc, 0)

    @pl.when(s == _NS - 1)
    def _zero_tail():
        pltpu.sync_copy(zbuf, acc_sh.at[pl.ds(_NS * _RPT, N - _NS * _RPT)])

    plsc.subcore_barrier()

    ebase = wid * _EPW

    def _body(i, carry):
        base = ebase + i * _CHUNK
        pltpu.sync_copy(src_hbm.at[pl.ds(base, _CHUNK)], src_v)
        pltpu.sync_copy(dst_hbm.at[pl.ds(base, _CHUNK)], dst_v)
        pltpu.async_copy(xw_hbm.at[src_v], rows_v, sem).wait()
        pltpu.sync_copy(rows_v, acc_sh.at[dst_v], add=True)
        return carry

    lax.fori_loop(0, _NCHUNK, _body, 0)
    plsc.subcore_barrier()

    pltpu.sync_copy(acc_sh.at[pl.ds(s * _RPT, _RPT)],
                    out_hbm.at[c, pl.ds(s * _RPT, _RPT)])

    @pl.when(s == _NS - 1)
    def _write_tail():
        pltpu.sync_copy(acc_sh.at[pl.ds(_NS * _RPT, N - _NS * _RPT)],
                        out_hbm.at[c, pl.ds(_NS * _RPT, N - _NS * _RPT)])


_sc_scatter = functools.partial(
    pl.kernel,
    out_type=jax.ShapeDtypeStruct((_NC, N, D), jnp.float32),
    mesh=plsc.VectorSubcoreMesh(core_axis_name="c", subcore_axis_name="s"),
    scratch_types=[
        pltpu.VMEM((_CHUNK,), jnp.int32),
        pltpu.VMEM((_CHUNK,), jnp.int32),
        pltpu.VMEM((_CHUNK, D), jnp.float32),
        pltpu.VMEM((_ZR, D), jnp.float32),
        pltpu.VMEM_SHARED((N, D), jnp.float32),
        pltpu.SemaphoreType.DMA,
    ],
)(_sc_scatter_body)


_BM = 2000


def _mm_body(x_ref, w_ref, o_ref):
    o_ref[...] = jnp.dot(x_ref[...], w_ref[...],
                         preferred_element_type=jnp.float32)


def _tc_mm(x, w):
    return pl.pallas_call(
        _mm_body,
        grid=(N // _BM,),
        in_specs=[
            pl.BlockSpec((_BM, D), lambda i: (i, 0)),
            pl.BlockSpec((D, D), lambda i: (0, 0)),
        ],
        out_specs=pl.BlockSpec((_BM, D), lambda i: (i, 0)),
        out_shape=jax.ShapeDtypeStruct((N, D), jnp.float32),
    )(x, w)


def _fuse_body(a0_ref, a1_ref, b_ref, w_ref, o_ref):
    h = jnp.maximum(a0_ref[...] + a1_ref[...] + b_ref[...], 0.0)
    o_ref[...] = jnp.dot(h, w_ref[...], preferred_element_type=jnp.float32)


def _tc_relu_mm(a0, a1, b, w):
    return pl.pallas_call(
        _fuse_body,
        grid=(N // _BM,),
        in_specs=[
            pl.BlockSpec((_BM, D), lambda i: (i, 0)),
            pl.BlockSpec((_BM, D), lambda i: (i, 0)),
            pl.BlockSpec((1, D), lambda i: (0, 0)),
            pl.BlockSpec((D, D), lambda i: (0, 0)),
        ],
        out_specs=pl.BlockSpec((_BM, D), lambda i: (i, 0)),
        out_shape=jax.ShapeDtypeStruct((N, D), jnp.float32),
    )(a0, a1, b, w)


def _final_body(a0_ref, a1_ref, b_ref, w_ref, bo_ref, o_ref):
    h = jnp.maximum(a0_ref[...] + a1_ref[...] + b_ref[...], 0.0)
    z = jnp.dot(h, w_ref[...], preferred_element_type=jnp.float32)
    o_ref[...] = jax.nn.sigmoid(z + bo_ref[...])


def _tc_final(a0, a1, b, w, bo):
    return pl.pallas_call(
        _final_body,
        grid=(N // _BM,),
        in_specs=[
            pl.BlockSpec((_BM, D), lambda i: (i, 0)),
            pl.BlockSpec((_BM, D), lambda i: (i, 0)),
            pl.BlockSpec((1, D), lambda i: (0, 0)),
            pl.BlockSpec((D, C), lambda i: (0, 0)),
            pl.BlockSpec((1, C), lambda i: (0, 0)),
        ],
        out_specs=pl.BlockSpec((_BM, C), lambda i: (i, 0)),
        out_shape=jax.ShapeDtypeStruct((N, C), jnp.float32),
    )(a0, a1, b, w, bo)


def kernel(F_, X, edge_index, Wf1, bf1, Wh1, bh1, Wf2, bf2, Wh2, bh2,
           Wout, bout):
    src = edge_index[0]
    dst = edge_index[1]
    bh1_2d = bh1.reshape(1, D)
    bh2_2d = bh2.reshape(1, D)
    bout_2d = bout.reshape(1, C)

    xw1 = _tc_mm(X, Wh1)
    acc1 = _sc_scatter(xw1, src, dst)
    xw2 = _tc_relu_mm(acc1[0], acc1[1], bh1_2d, Wh2)
    acc2 = _sc_scatter(xw2, src, dst)
    return _tc_final(acc2[0], acc2[1], bh2_2d, Wout, bout_2d)
